# Initial kernel scaffold; baseline (speedup 1.0000x reference)
import jax
import jax.numpy as jnp
from jax.experimental import pallas as pl


def kernel(input_ids, embed_tokens, embed_positions, in_proj_w, in_proj_b, out_proj_w, out_proj_b, ln1_w, ln1_b, lin1_w, lin1_b, lin2_w, lin2_b, ln2_w, ln2_b, router_w, expert_w1, expert_b1, expert_w2, expert_b2, lnf_w, lnf_b):
    raise NotImplementedError("write your pallas kernel here")


# speedup vs baseline: 1.7928x; 1.7928x over previous
import functools

import jax
import jax.numpy as jnp
from jax import lax
from jax.experimental import pallas as pl
from jax.experimental.pallas import tpu as pltpu
from jax.experimental.pallas import tpu_sc as plsc

S = 2048
D = 768
H = 12
DH = 64
E = 8
FF = 3072
EFF = 1024
EPAD = 128

RB = 512
NRB = S // RB


SC_NC = 2
SC_NS = 16


def _sc_embed_gather(ids, table):
    nw = SC_NC * SC_NS
    bpw = S // nw
    mesh = plsc.VectorSubcoreMesh(core_axis_name="c", subcore_axis_name="s",
                                  num_cores=SC_NC, num_subcores=SC_NS)

    @functools.partial(
        pl.kernel,
        mesh=mesh,
        out_type=jax.ShapeDtypeStruct((S, D), jnp.float32),
        scratch_types=[
            pltpu.VMEM((bpw,), jnp.int32),
            pltpu.VMEM((bpw, D), jnp.float32),
            pltpu.SemaphoreType.DMA,
        ],
    )
    def k(ids_hbm, tab_hbm, out_hbm, idx_v, rows_v, sem):
        wid = lax.axis_index("s") * SC_NC + lax.axis_index("c")
        base = wid * bpw
        pltpu.sync_copy(ids_hbm.at[pl.ds(base, bpw)], idx_v)
        pltpu.async_copy(tab_hbm.at[idx_v], rows_v, sem).wait()
        pltpu.sync_copy(rows_v, out_hbm.at[pl.ds(base, bpw)])

    return k(ids, table)


def _qkv_body(emb_ref, pos_ref, w_ref, b_ref, h_ref, qkv_ref):
    h = emb_ref[...] + pos_ref[...]
    h_ref[...] = h
    qkv_ref[...] = lax.dot_general(
        h, w_ref[...], (((1,), (1,)), ((), ())),
        preferred_element_type=jnp.float32) + b_ref[...]


def _qkv(emb, pos, w_t, b):
    return pl.pallas_call(
        _qkv_body,
        grid=(NRB,),
        in_specs=[
            pl.BlockSpec((RB, D), lambda i: (i, 0)),
            pl.BlockSpec((RB, D), lambda i: (i, 0)),
            pl.BlockSpec((3 * D, D), lambda i: (0, 0)),
            pl.BlockSpec((1, 3 * D), lambda i: (0, 0)),
        ],
        out_specs=[
            pl.BlockSpec((RB, D), lambda i: (i, 0)),
            pl.BlockSpec((RB, 3 * D), lambda i: (i, 0)),
        ],
        out_shape=[
            jax.ShapeDtypeStruct((S, D), jnp.float32),
            jax.ShapeDtypeStruct((S, 3 * D), jnp.float32),
        ],
    )(emb, pos, w_t, b)


ACH = 1024


def _attn_body(q_ref, k_ref, v_ref, o_ref):
    q = q_ref[0]
    m = jnp.full((S, 1), -jnp.inf, jnp.float32)
    den = jnp.zeros((S, 1), jnp.float32)
    acc = jnp.zeros((S, DH), jnp.float32)
    for c in range(S // ACH):
        kc = k_ref[0, c * ACH:(c + 1) * ACH, :]
        vc = v_ref[0, c * ACH:(c + 1) * ACH, :]
        s = lax.dot_general(q, kc, (((1,), (1,)), ((), ())),
                            preferred_element_type=jnp.float32)
        s = s * (1.0 / (DH ** 0.5))
        mn = jnp.maximum(m, jnp.max(s, axis=-1, keepdims=True))
        alpha = jnp.exp(m - mn)
        e = jnp.exp(s - mn)
        den = den * alpha + jnp.sum(e, axis=-1, keepdims=True)
        acc = acc * alpha + jnp.dot(e, vc, preferred_element_type=jnp.float32)
        m = mn
    o_ref[0] = acc / den


def _attention(q3, k3, v3):
    return pl.pallas_call(
        _attn_body,
        grid=(H,),
        in_specs=[
            pl.BlockSpec((1, S, DH), lambda h: (h, 0, 0)),
            pl.BlockSpec((1, S, DH), lambda h: (h, 0, 0)),
            pl.BlockSpec((1, S, DH), lambda h: (h, 0, 0)),
        ],
        out_specs=pl.BlockSpec((1, S, DH), lambda h: (h, 0, 0)),
        out_shape=jax.ShapeDtypeStruct((H, S, DH), jnp.float32),
    )(q3, k3, v3)


def _ln(x, w, b):
    m = jnp.mean(x, axis=-1, keepdims=True)
    xc = x - m
    v = jnp.mean(xc * xc, axis=-1, keepdims=True)
    return xc / jnp.sqrt(v + 1e-5) * w + b


def _proj_ln_body(a_ref, w_ref, b_ref, h_ref, lw_ref, lb_ref, o_ref):
    o = lax.dot_general(
        a_ref[...], w_ref[...], (((1,), (1,)), ((), ())),
        preferred_element_type=jnp.float32) + b_ref[...]
    o_ref[...] = _ln(h_ref[...] + o, lw_ref[...], lb_ref[...])


def _proj_ln(attn, w_t, b, h, lw, lb):
    return pl.pallas_call(
        _proj_ln_body,
        grid=(NRB,),
        in_specs=[
            pl.BlockSpec((RB, D), lambda i: (i, 0)),
            pl.BlockSpec((D, D), lambda i: (0, 0)),
            pl.BlockSpec((1, D), lambda i: (0, 0)),
            pl.BlockSpec((RB, D), lambda i: (i, 0)),
            pl.BlockSpec((1, D), lambda i: (0, 0)),
            pl.BlockSpec((1, D), lambda i: (0, 0)),
        ],
        out_specs=pl.BlockSpec((RB, D), lambda i: (i, 0)),
        out_shape=jax.ShapeDtypeStruct((S, D), jnp.float32),
    )(attn, w_t, b, h, lw, lb)


def _ffn_body(h_ref, w1_ref, b1_ref, w2_ref, b2_ref, lw_ref, lb_ref, o_ref):
    h = h_ref[...]
    mid = jnp.maximum(
        lax.dot_general(h, w1_ref[...], (((1,), (1,)), ((), ())),
                        preferred_element_type=jnp.float32)
        + b1_ref[...], 0.0)
    ff = lax.dot_general(mid, w2_ref[...], (((1,), (1,)), ((), ())),
                         preferred_element_type=jnp.float32) + b2_ref[...]
    o_ref[...] = _ln(h + ff, lw_ref[...], lb_ref[...])


def _ffn(h, w1_t, b1, w2_t, b2, lw, lb):
    return pl.pallas_call(
        _ffn_body,
        grid=(NRB,),
        in_specs=[
            pl.BlockSpec((RB, D), lambda i: (i, 0)),
            pl.BlockSpec((FF, D), lambda i: (0, 0)),
            pl.BlockSpec((1, FF), lambda i: (0, 0)),
            pl.BlockSpec((D, FF), lambda i: (0, 0)),
            pl.BlockSpec((1, D), lambda i: (0, 0)),
            pl.BlockSpec((1, D), lambda i: (0, 0)),
            pl.BlockSpec((1, D), lambda i: (0, 0)),
        ],
        out_specs=pl.BlockSpec((RB, D), lambda i: (i, 0)),
        out_shape=jax.ShapeDtypeStruct((S, D), jnp.float32),
    )(h, w1_t, b1, w2_t, b2, lw, lb)


def _router_body(x_ref, w_ref, comb_ref, loss_ref):
    lp = lax.dot_general(x_ref[...], w_ref[...], (((1,), (1,)), ((), ())),
                         preferred_element_type=jnp.float32)
    col = lax.broadcasted_iota(jnp.int32, (S, EPAD), 1)
    valid = col < E
    neg = jnp.float32(-1e30)
    lpm = jnp.where(valid, lp, neg)
    m = jnp.max(lpm, axis=-1, keepdims=True)
    el = jnp.where(valid, jnp.exp(lpm - m), 0.0)
    denom = jnp.sum(el, axis=-1, keepdims=True)
    p = el / denom
    m1 = jnp.max(p, axis=-1, keepdims=True)
    i1 = jnp.min(jnp.where((p == m1) & valid, col, EPAD), axis=-1,
                 keepdims=True)
    p2 = jnp.where(col == i1, -1.0, p)
    m2 = jnp.max(p2, axis=-1, keepdims=True)
    i2 = jnp.min(jnp.where((p2 == m2) & valid, col, EPAD), axis=-1,
                 keepdims=True)
    wsum = m1 + m2
    sel1 = (col == i1).astype(jnp.float32)
    sel2 = (col == i2).astype(jnp.float32)
    comb_ref[...] = (sel1 * (m1 / wsum) + sel2 * (m2 / wsum))
    counts = jnp.sum(sel1 + sel2, axis=0, keepdims=True)
    load = counts / jnp.float32(S * 2)
    dev = jnp.where(col[:1] < E, load - 1.0 / E, 0.0)
    aux = jnp.sum(dev * dev) / jnp.float32(E)
    zrow = jnp.log(denom) + m
    zl = jnp.sum(zrow) / jnp.float32(S)
    loss_ref[...] = jnp.full((8, 128), 0.01 * aux + 0.001 * zl,
                             dtype=jnp.float32)


def _router(x, wr_pad_t):
    return pl.pallas_call(
        _router_body,
        in_specs=[
            pl.BlockSpec((S, D), lambda: (0, 0)),
            pl.BlockSpec((EPAD, D), lambda: (0, 0)),
        ],
        out_specs=[
            pl.BlockSpec((S, EPAD), lambda: (0, 0)),
            pl.BlockSpec((8, 128), lambda: (0, 0)),
        ],
        out_shape=[
            jax.ShapeDtypeStruct((S, EPAD), jnp.float32),
            jax.ShapeDtypeStruct((8, 128), jnp.float32),
        ],
    )(x, wr_pad_t)


def _moe_body(x_ref, w1_ref, b1_ref, w2_ref, b2_ref, comb_ref, o_ref):
    e = pl.program_id(1)
    x = x_ref[...]
    mid = jnp.dot(x, w1_ref[0], preferred_element_type=jnp.float32) \
        + b1_ref[0]
    mid = 0.5 * mid * (1.0 + lax.erf(mid * (2.0 ** -0.5)))
    y = jnp.dot(mid, w2_ref[0], preferred_element_type=jnp.float32) \
        + b2_ref[0]
    col = lax.broadcasted_iota(jnp.int32, (RB, EPAD), 1)
    cw = jnp.sum(jnp.where(col == e, comb_ref[...], 0.0), axis=-1,
                 keepdims=True)
    y = y * cw

    @pl.when(e == 0)
    def _():
        o_ref[...] = y

    @pl.when(e > 0)
    def _():
        o_ref[...] += y


def _moe(x, ew1_t, eb1, ew2_t, eb2, comb):
    return pl.pallas_call(
        _moe_body,
        grid=(NRB, E),
        in_specs=[
            pl.BlockSpec((RB, D), lambda i, e: (i, 0)),
            pl.BlockSpec((1, D, EFF), lambda i, e: (e, 0, 0)),
            pl.BlockSpec((1, 1, EFF), lambda i, e: (e, 0, 0)),
            pl.BlockSpec((1, EFF, D), lambda i, e: (e, 0, 0)),
            pl.BlockSpec((1, 1, D), lambda i, e: (e, 0, 0)),
            pl.BlockSpec((RB, EPAD), lambda i, e: (i, 0)),
        ],
        out_specs=pl.BlockSpec((RB, D), lambda i, e: (i, 0)),
        out_shape=jax.ShapeDtypeStruct((S, D), jnp.float32),
    )(x, ew1_t, eb1, ew2_t, eb2, comb)


def _lnf_body(x_ref, lw_ref, lb_ref, o_ref):
    o_ref[...] = _ln(x_ref[...], lw_ref[...], lb_ref[...])


def _lnf(x, lw, lb):
    return pl.pallas_call(
        _lnf_body,
        grid=(NRB,),
        in_specs=[
            pl.BlockSpec((RB, D), lambda i: (i, 0)),
            pl.BlockSpec((1, D), lambda i: (0, 0)),
            pl.BlockSpec((1, D), lambda i: (0, 0)),
        ],
        out_specs=pl.BlockSpec((RB, D), lambda i: (i, 0)),
        out_shape=jax.ShapeDtypeStruct((S, D), jnp.float32),
    )(x, lw, lb)


def kernel(input_ids, embed_tokens, embed_positions, in_proj_w, in_proj_b,
           out_proj_w, out_proj_b, ln1_w, ln1_b, lin1_w, lin1_b, lin2_w,
           lin2_b, ln2_w, ln2_b, router_w, expert_w1, expert_b1, expert_w2,
           expert_b2, lnf_w, lnf_b):
    ids = input_ids.reshape(-1).astype(jnp.int32)
    emb = _sc_embed_gather(ids, embed_tokens)

    row = lambda v: v.reshape(1, -1)
    h, qkv = _qkv(emb, embed_positions[:S], in_proj_w, row(in_proj_b))
    hd = lambda t: t.reshape(S, H, DH).transpose(1, 0, 2)
    attn3 = _attention(hd(qkv[:, :D]), hd(qkv[:, D:2 * D]),
                       hd(qkv[:, 2 * D:]))
    attn = attn3.transpose(1, 0, 2).reshape(S, D)
    h1 = _proj_ln(attn, out_proj_w, row(out_proj_b), h,
                  row(ln1_w), row(ln1_b))
    x = _ffn(h1, lin1_w, row(lin1_b), lin2_w, row(lin2_b),
             row(ln2_w), row(ln2_b))

    wr_pad_t = jnp.zeros((EPAD, D), jnp.float32).at[:E].set(router_w)
    comb, loss = _router(x, wr_pad_t)

    ew1_t = expert_w1.transpose(0, 2, 1)
    ew2_t = expert_w2.transpose(0, 2, 1)
    out = _moe(x, ew1_t, expert_b1.reshape(E, 1, EFF), ew2_t,
               expert_b2.reshape(E, 1, D), comb)

    hf = _lnf(out, row(lnf_w), row(lnf_b))
    return hf.reshape(1, S, D), loss[0, 0]


# speedup vs baseline: 1.9076x; 1.0640x over previous
import functools

import jax
import jax.numpy as jnp
from jax import lax
from jax.experimental import pallas as pl
from jax.experimental.pallas import tpu as pltpu
from jax.experimental.pallas import tpu_sc as plsc

S = 2048
D = 768
H = 12
DH = 64
E = 8
FF = 3072
EFF = 1024
EPAD = 128

RB = 512
NRB = S // RB


SC_NC = 2
SC_NS = 16


def _sc_embed_gather(ids, table):
    nw = SC_NC * SC_NS
    bpw = S // nw
    mesh = plsc.VectorSubcoreMesh(core_axis_name="c", subcore_axis_name="s",
                                  num_cores=SC_NC, num_subcores=SC_NS)

    @functools.partial(
        pl.kernel,
        mesh=mesh,
        out_type=jax.ShapeDtypeStruct((S, D), jnp.float32),
        scratch_types=[
            pltpu.VMEM((bpw,), jnp.int32),
            pltpu.VMEM((bpw, D), jnp.float32),
            pltpu.SemaphoreType.DMA,
        ],
    )
    def k(ids_hbm, tab_hbm, out_hbm, idx_v, rows_v, sem):
        wid = lax.axis_index("s") * SC_NC + lax.axis_index("c")
        base = wid * bpw
        pltpu.sync_copy(ids_hbm.at[pl.ds(base, bpw)], idx_v)
        pltpu.async_copy(tab_hbm.at[idx_v], rows_v, sem).wait()
        pltpu.sync_copy(rows_v, out_hbm.at[pl.ds(base, bpw)])

    return k(ids, table)


def _qkv_body(emb_ref, pos_ref, w_ref, b_ref, h_ref, qkv_ref):
    h = emb_ref[...] + pos_ref[...]
    h_ref[...] = h
    qkv_ref[...] = lax.dot_general(
        h, w_ref[...], (((1,), (1,)), ((), ())),
        preferred_element_type=jnp.float32) + b_ref[...]


def _qkv(emb, pos, w_t, b):
    return pl.pallas_call(
        _qkv_body,
        grid=(NRB,),
        in_specs=[
            pl.BlockSpec((RB, D), lambda i: (i, 0)),
            pl.BlockSpec((RB, D), lambda i: (i, 0)),
            pl.BlockSpec((3 * D, D), lambda i: (0, 0)),
            pl.BlockSpec((1, 3 * D), lambda i: (0, 0)),
        ],
        out_specs=[
            pl.BlockSpec((RB, D), lambda i: (i, 0)),
            pl.BlockSpec((RB, 3 * D), lambda i: (i, 0)),
        ],
        out_shape=[
            jax.ShapeDtypeStruct((S, D), jnp.float32),
            jax.ShapeDtypeStruct((S, 3 * D), jnp.float32),
        ],
    )(emb, pos, w_t, b)


ACH = 1024


def _attn_body(q_ref, k_ref, v_ref, o_ref):
    q = q_ref[0]
    m = jnp.full((S, 1), -jnp.inf, jnp.float32)
    den = jnp.zeros((S, 1), jnp.float32)
    acc = jnp.zeros((S, DH), jnp.float32)
    for c in range(S // ACH):
        kc = k_ref[0, c * ACH:(c + 1) * ACH, :]
        vc = v_ref[0, c * ACH:(c + 1) * ACH, :]
        s = lax.dot_general(q, kc, (((1,), (1,)), ((), ())),
                            preferred_element_type=jnp.float32)
        s = s * (1.0 / (DH ** 0.5))
        mn = jnp.maximum(m, jnp.max(s, axis=-1, keepdims=True))
        alpha = jnp.exp(m - mn)
        e = jnp.exp(s - mn)
        den = den * alpha + jnp.sum(e, axis=-1, keepdims=True)
        acc = acc * alpha + jnp.dot(e, vc, preferred_element_type=jnp.float32)
        m = mn
    o_ref[0] = acc / den


def _attention(q3, k3, v3):
    return pl.pallas_call(
        _attn_body,
        grid=(H,),
        in_specs=[
            pl.BlockSpec((1, S, DH), lambda h: (h, 0, 0)),
            pl.BlockSpec((1, S, DH), lambda h: (h, 0, 0)),
            pl.BlockSpec((1, S, DH), lambda h: (h, 0, 0)),
        ],
        out_specs=pl.BlockSpec((1, S, DH), lambda h: (h, 0, 0)),
        out_shape=jax.ShapeDtypeStruct((H, S, DH), jnp.float32),
    )(q3, k3, v3)


def _ln(x, w, b):
    m = jnp.mean(x, axis=-1, keepdims=True)
    xc = x - m
    v = jnp.mean(xc * xc, axis=-1, keepdims=True)
    return xc / jnp.sqrt(v + 1e-5) * w + b


def _proj_ln_body(a_ref, w_ref, b_ref, h_ref, lw_ref, lb_ref, o_ref):
    o = lax.dot_general(
        a_ref[...], w_ref[...], (((1,), (1,)), ((), ())),
        preferred_element_type=jnp.float32) + b_ref[...]
    o_ref[...] = _ln(h_ref[...] + o, lw_ref[...], lb_ref[...])


def _proj_ln(attn, w_t, b, h, lw, lb):
    return pl.pallas_call(
        _proj_ln_body,
        grid=(NRB,),
        in_specs=[
            pl.BlockSpec((RB, D), lambda i: (i, 0)),
            pl.BlockSpec((D, D), lambda i: (0, 0)),
            pl.BlockSpec((1, D), lambda i: (0, 0)),
            pl.BlockSpec((RB, D), lambda i: (i, 0)),
            pl.BlockSpec((1, D), lambda i: (0, 0)),
            pl.BlockSpec((1, D), lambda i: (0, 0)),
        ],
        out_specs=pl.BlockSpec((RB, D), lambda i: (i, 0)),
        out_shape=jax.ShapeDtypeStruct((S, D), jnp.float32),
    )(attn, w_t, b, h, lw, lb)


def _ffn_body(h_ref, w1_ref, b1_ref, w2_ref, b2_ref, lw_ref, lb_ref, o_ref):
    h = h_ref[...]
    mid = jnp.maximum(
        lax.dot_general(h, w1_ref[...], (((1,), (1,)), ((), ())),
                        preferred_element_type=jnp.float32)
        + b1_ref[...], 0.0)
    ff = lax.dot_general(mid, w2_ref[...], (((1,), (1,)), ((), ())),
                         preferred_element_type=jnp.float32) + b2_ref[...]
    o_ref[...] = _ln(h + ff, lw_ref[...], lb_ref[...])


def _ffn(h, w1_t, b1, w2_t, b2, lw, lb):
    return pl.pallas_call(
        _ffn_body,
        grid=(NRB,),
        in_specs=[
            pl.BlockSpec((RB, D), lambda i: (i, 0)),
            pl.BlockSpec((FF, D), lambda i: (0, 0)),
            pl.BlockSpec((1, FF), lambda i: (0, 0)),
            pl.BlockSpec((D, FF), lambda i: (0, 0)),
            pl.BlockSpec((1, D), lambda i: (0, 0)),
            pl.BlockSpec((1, D), lambda i: (0, 0)),
            pl.BlockSpec((1, D), lambda i: (0, 0)),
        ],
        out_specs=pl.BlockSpec((RB, D), lambda i: (i, 0)),
        out_shape=jax.ShapeDtypeStruct((S, D), jnp.float32),
    )(h, w1_t, b1, w2_t, b2, lw, lb)


def _router_body(x_ref, w_ref, comb_ref, loss_ref):
    lp = lax.dot_general(x_ref[...], w_ref[...], (((1,), (1,)), ((), ())),
                         preferred_element_type=jnp.float32)
    col = lax.broadcasted_iota(jnp.int32, (S, EPAD), 1)
    valid = col < E
    neg = jnp.float32(-1e30)
    lpm = jnp.where(valid, lp, neg)
    m = jnp.max(lpm, axis=-1, keepdims=True)
    el = jnp.where(valid, jnp.exp(lpm - m), 0.0)
    denom = jnp.sum(el, axis=-1, keepdims=True)
    p = el / denom
    m1 = jnp.max(p, axis=-1, keepdims=True)
    i1 = jnp.min(jnp.where((p == m1) & valid, col, EPAD), axis=-1,
                 keepdims=True)
    p2 = jnp.where(col == i1, -1.0, p)
    m2 = jnp.max(p2, axis=-1, keepdims=True)
    i2 = jnp.min(jnp.where((p2 == m2) & valid, col, EPAD), axis=-1,
                 keepdims=True)
    wsum = m1 + m2
    sel1 = (col == i1).astype(jnp.float32)
    sel2 = (col == i2).astype(jnp.float32)
    comb_ref[...] = (sel1 * (m1 / wsum) + sel2 * (m2 / wsum))
    counts = jnp.sum(sel1 + sel2, axis=0, keepdims=True)
    load = counts / jnp.float32(S * 2)
    dev = jnp.where(col[:1] < E, load - 1.0 / E, 0.0)
    aux = jnp.sum(dev * dev) / jnp.float32(E)
    zrow = jnp.log(denom) + m
    zl = jnp.sum(zrow) / jnp.float32(S)
    loss_ref[...] = jnp.full((8, 128), 0.01 * aux + 0.001 * zl,
                             dtype=jnp.float32)


def _router(x, wr_pad_t):
    return pl.pallas_call(
        _router_body,
        in_specs=[
            pl.BlockSpec((S, D), lambda: (0, 0)),
            pl.BlockSpec((EPAD, D), lambda: (0, 0)),
        ],
        out_specs=[
            pl.BlockSpec((S, EPAD), lambda: (0, 0)),
            pl.BlockSpec((8, 128), lambda: (0, 0)),
        ],
        out_shape=[
            jax.ShapeDtypeStruct((S, EPAD), jnp.float32),
            jax.ShapeDtypeStruct((8, 128), jnp.float32),
        ],
    )(x, wr_pad_t)


def _moe_body(x_ref, w1_ref, b1_ref, w2_ref, b2_ref, comb_ref, lw_ref,
              lb_ref, o_ref):
    e = pl.program_id(0)
    x = x_ref[...]
    mid = jnp.dot(x, w1_ref[0], preferred_element_type=jnp.float32) \
        + b1_ref[0]
    mid = 0.5 * mid * (1.0 + lax.erf(mid * (2.0 ** -0.5)))
    y = jnp.dot(mid, w2_ref[0], preferred_element_type=jnp.float32) \
        + b2_ref[0]
    col = lax.broadcasted_iota(jnp.int32, (S, EPAD), 1)
    cw = jnp.sum(jnp.where(col == e, comb_ref[...], 0.0), axis=-1,
                 keepdims=True)
    y = y * cw

    @pl.when(e == 0)
    def _():
        o_ref[...] = y

    @pl.when(e > 0)
    def _():
        o_ref[...] += y

    @pl.when(e == E - 1)
    def _():
        o_ref[...] = _ln(o_ref[...], lw_ref[...], lb_ref[...])


def _moe(x, ew1_t, eb1, ew2_t, eb2, comb, lw, lb):
    return pl.pallas_call(
        _moe_body,
        grid=(E,),
        in_specs=[
            pl.BlockSpec((S, D), lambda e: (0, 0)),
            pl.BlockSpec((1, D, EFF), lambda e: (e, 0, 0)),
            pl.BlockSpec((1, 1, EFF), lambda e: (e, 0, 0)),
            pl.BlockSpec((1, EFF, D), lambda e: (e, 0, 0)),
            pl.BlockSpec((1, 1, D), lambda e: (e, 0, 0)),
            pl.BlockSpec((S, EPAD), lambda e: (0, 0)),
            pl.BlockSpec((1, D), lambda e: (0, 0)),
            pl.BlockSpec((1, D), lambda e: (0, 0)),
        ],
        out_specs=pl.BlockSpec((S, D), lambda e: (0, 0)),
        out_shape=jax.ShapeDtypeStruct((S, D), jnp.float32),
    )(x, ew1_t, eb1, ew2_t, eb2, comb, lw, lb)


def kernel(input_ids, embed_tokens, embed_positions, in_proj_w, in_proj_b,
           out_proj_w, out_proj_b, ln1_w, ln1_b, lin1_w, lin1_b, lin2_w,
           lin2_b, ln2_w, ln2_b, router_w, expert_w1, expert_b1, expert_w2,
           expert_b2, lnf_w, lnf_b):
    ids = input_ids.reshape(-1).astype(jnp.int32)
    emb = _sc_embed_gather(ids, embed_tokens)

    row = lambda v: v.reshape(1, -1)
    h, qkv = _qkv(emb, embed_positions[:S], in_proj_w, row(in_proj_b))
    hd = lambda t: t.reshape(S, H, DH).transpose(1, 0, 2)
    attn3 = _attention(hd(qkv[:, :D]), hd(qkv[:, D:2 * D]),
                       hd(qkv[:, 2 * D:]))
    attn = attn3.transpose(1, 0, 2).reshape(S, D)
    h1 = _proj_ln(attn, out_proj_w, row(out_proj_b), h,
                  row(ln1_w), row(ln1_b))
    x = _ffn(h1, lin1_w, row(lin1_b), lin2_w, row(lin2_b),
             row(ln2_w), row(ln2_b))

    wr_pad_t = jnp.zeros((EPAD, D), jnp.float32).at[:E].set(router_w)
    comb, loss = _router(x, wr_pad_t)

    ew1_t = expert_w1.transpose(0, 2, 1)
    ew2_t = expert_w2.transpose(0, 2, 1)
    hf = _moe(x, ew1_t, expert_b1.reshape(E, 1, EFF), ew2_t,
              expert_b2.reshape(E, 1, D), comb, row(lnf_w), row(lnf_b))
    return hf.reshape(1, S, D), loss[0, 0]


# speedup vs baseline: 1.9571x; 1.0259x over previous
import functools

import jax
import jax.numpy as jnp
from jax import lax
from jax.experimental import pallas as pl
from jax.experimental.pallas import tpu as pltpu
from jax.experimental.pallas import tpu_sc as plsc

S = 2048
D = 768
H = 12
DH = 64
E = 8
FF = 3072
EFF = 1024
EPAD = 128

RB = 512
NRB = S // RB


SC_NC = 2
SC_NS = 16


def _sc_embed_gather(ids, table):
    nw = SC_NC * SC_NS
    bpw = S // nw
    mesh = plsc.VectorSubcoreMesh(core_axis_name="c", subcore_axis_name="s",
                                  num_cores=SC_NC, num_subcores=SC_NS)

    @functools.partial(
        pl.kernel,
        mesh=mesh,
        out_type=jax.ShapeDtypeStruct((S, D), jnp.float32),
        scratch_types=[
            pltpu.VMEM((bpw,), jnp.int32),
            pltpu.VMEM((bpw, D), jnp.float32),
            pltpu.SemaphoreType.DMA,
        ],
    )
    def k(ids_hbm, tab_hbm, out_hbm, idx_v, rows_v, sem):
        wid = lax.axis_index("s") * SC_NC + lax.axis_index("c")
        base = wid * bpw
        pltpu.sync_copy(ids_hbm.at[pl.ds(base, bpw)], idx_v)
        pltpu.async_copy(tab_hbm.at[idx_v], rows_v, sem).wait()
        pltpu.sync_copy(rows_v, out_hbm.at[pl.ds(base, bpw)])

    return k(ids, table)


def _qkv_body(emb_ref, pos_ref, w_ref, b_ref, h_ref, qkv_ref):
    h = emb_ref[...] + pos_ref[...]
    h_ref[...] = h
    qkv_ref[...] = lax.dot_general(
        h, w_ref[...], (((1,), (1,)), ((), ())),
        preferred_element_type=jnp.float32) + b_ref[...]


def _qkv(emb, pos, w_t, b):
    return pl.pallas_call(
        _qkv_body,
        grid=(NRB,),
        in_specs=[
            pl.BlockSpec((RB, D), lambda i: (i, 0)),
            pl.BlockSpec((RB, D), lambda i: (i, 0)),
            pl.BlockSpec((3 * D, D), lambda i: (0, 0)),
            pl.BlockSpec((1, 3 * D), lambda i: (0, 0)),
        ],
        out_specs=[
            pl.BlockSpec((RB, D), lambda i: (i, 0)),
            pl.BlockSpec((RB, 3 * D), lambda i: (i, 0)),
        ],
        out_shape=[
            jax.ShapeDtypeStruct((S, D), jnp.float32),
            jax.ShapeDtypeStruct((S, 3 * D), jnp.float32),
        ],
    )(emb, pos, w_t, b)


ACH = 1024


def _attn_body(q_ref, k_ref, v_ref, o_ref):
    q = q_ref[0]
    m = jnp.full((S, 1), -jnp.inf, jnp.float32)
    den = jnp.zeros((S, 1), jnp.float32)
    acc = jnp.zeros((S, DH), jnp.float32)
    for c in range(S // ACH):
        kc = k_ref[0, c * ACH:(c + 1) * ACH, :]
        vc = v_ref[0, c * ACH:(c + 1) * ACH, :]
        s = lax.dot_general(q, kc, (((1,), (1,)), ((), ())),
                            preferred_element_type=jnp.float32)
        s = s * (1.0 / (DH ** 0.5))
        mn = jnp.maximum(m, jnp.max(s, axis=-1, keepdims=True))
        alpha = jnp.exp(m - mn)
        e = jnp.exp(s - mn)
        den = den * alpha + jnp.sum(e, axis=-1, keepdims=True)
        acc = acc * alpha + jnp.dot(e, vc, preferred_element_type=jnp.float32)
        m = mn
    o_ref[0] = acc / den


def _attention(q3, k3, v3):
    return pl.pallas_call(
        _attn_body,
        grid=(H,),
        in_specs=[
            pl.BlockSpec((1, S, DH), lambda h: (h, 0, 0)),
            pl.BlockSpec((1, S, DH), lambda h: (h, 0, 0)),
            pl.BlockSpec((1, S, DH), lambda h: (h, 0, 0)),
        ],
        out_specs=pl.BlockSpec((1, S, DH), lambda h: (h, 0, 0)),
        out_shape=jax.ShapeDtypeStruct((H, S, DH), jnp.float32),
    )(q3, k3, v3)


def _ln(x, w, b):
    m = jnp.mean(x, axis=-1, keepdims=True)
    xc = x - m
    v = jnp.mean(xc * xc, axis=-1, keepdims=True)
    return xc / jnp.sqrt(v + 1e-5) * w + b


def _tail_body(a_ref, w_ref, b_ref, h_ref, l1w_ref, l1b_ref, w1_ref, b1_ref,
               w2_ref, b2_ref, l2w_ref, l2b_ref, o_ref):
    o = lax.dot_general(
        a_ref[...], w_ref[...], (((1,), (1,)), ((), ())),
        preferred_element_type=jnp.float32) + b_ref[...]
    h1 = _ln(h_ref[...] + o, l1w_ref[...], l1b_ref[...])
    mid = jnp.maximum(
        lax.dot_general(h1, w1_ref[...], (((1,), (1,)), ((), ())),
                        preferred_element_type=jnp.float32)
        + b1_ref[...], 0.0)
    ff = lax.dot_general(mid, w2_ref[...], (((1,), (1,)), ((), ())),
                         preferred_element_type=jnp.float32) + b2_ref[...]
    o_ref[...] = _ln(h1 + ff, l2w_ref[...], l2b_ref[...])


def _layer_tail(attn, w_t, b, h, l1w, l1b, w1_t, b1, w2_t, b2, l2w, l2b):
    rowspec = pl.BlockSpec((1, D), lambda i: (0, 0))
    return pl.pallas_call(
        _tail_body,
        grid=(NRB,),
        in_specs=[
            pl.BlockSpec((RB, D), lambda i: (i, 0)),
            pl.BlockSpec((D, D), lambda i: (0, 0)),
            rowspec,
            pl.BlockSpec((RB, D), lambda i: (i, 0)),
            rowspec,
            rowspec,
            pl.BlockSpec((FF, D), lambda i: (0, 0)),
            pl.BlockSpec((1, FF), lambda i: (0, 0)),
            pl.BlockSpec((D, FF), lambda i: (0, 0)),
            rowspec,
            rowspec,
            rowspec,
        ],
        out_specs=pl.BlockSpec((RB, D), lambda i: (i, 0)),
        out_shape=jax.ShapeDtypeStruct((S, D), jnp.float32),
    )(attn, w_t, b, h, l1w, l1b, w1_t, b1, w2_t, b2, l2w, l2b)


def _router_body(x_ref, w_ref, comb_ref, loss_ref):
    lp = lax.dot_general(x_ref[...], w_ref[...], (((1,), (1,)), ((), ())),
                         preferred_element_type=jnp.float32)
    col = lax.broadcasted_iota(jnp.int32, (S, EPAD), 1)
    valid = col < E
    neg = jnp.float32(-1e30)
    lpm = jnp.where(valid, lp, neg)
    m = jnp.max(lpm, axis=-1, keepdims=True)
    el = jnp.where(valid, jnp.exp(lpm - m), 0.0)
    denom = jnp.sum(el, axis=-1, keepdims=True)
    p = el / denom
    m1 = jnp.max(p, axis=-1, keepdims=True)
    i1 = jnp.min(jnp.where((p == m1) & valid, col, EPAD), axis=-1,
                 keepdims=True)
    p2 = jnp.where(col == i1, -1.0, p)
    m2 = jnp.max(p2, axis=-1, keepdims=True)
    i2 = jnp.min(jnp.where((p2 == m2) & valid, col, EPAD), axis=-1,
                 keepdims=True)
    wsum = m1 + m2
    sel1 = (col == i1).astype(jnp.float32)
    sel2 = (col == i2).astype(jnp.float32)
    comb_ref[...] = (sel1 * (m1 / wsum) + sel2 * (m2 / wsum))
    counts = jnp.sum(sel1 + sel2, axis=0, keepdims=True)
    load = counts / jnp.float32(S * 2)
    dev = jnp.where(col[:1] < E, load - 1.0 / E, 0.0)
    aux = jnp.sum(dev * dev) / jnp.float32(E)
    zrow = jnp.log(denom) + m
    zl = jnp.sum(zrow) / jnp.float32(S)
    loss_ref[...] = jnp.full((8, 128), 0.01 * aux + 0.001 * zl,
                             dtype=jnp.float32)


def _router(x, wr_pad_t):
    return pl.pallas_call(
        _router_body,
        in_specs=[
            pl.BlockSpec((S, D), lambda: (0, 0)),
            pl.BlockSpec((EPAD, D), lambda: (0, 0)),
        ],
        out_specs=[
            pl.BlockSpec((S, EPAD), lambda: (0, 0)),
            pl.BlockSpec((8, 128), lambda: (0, 0)),
        ],
        out_shape=[
            jax.ShapeDtypeStruct((S, EPAD), jnp.float32),
            jax.ShapeDtypeStruct((8, 128), jnp.float32),
        ],
    )(x, wr_pad_t)


def _moe_body(x_ref, w1_ref, b1_ref, w2_ref, b2_ref, comb_ref, lw_ref,
              lb_ref, o_ref):
    e = pl.program_id(0)
    x = x_ref[...]
    mid = jnp.dot(x, w1_ref[0], preferred_element_type=jnp.float32) \
        + b1_ref[0]
    mid = 0.5 * mid * (1.0 + lax.erf(mid * (2.0 ** -0.5)))
    y = jnp.dot(mid, w2_ref[0], preferred_element_type=jnp.float32) \
        + b2_ref[0]
    col = lax.broadcasted_iota(jnp.int32, (S, EPAD), 1)
    cw = jnp.sum(jnp.where(col == e, comb_ref[...], 0.0), axis=-1,
                 keepdims=True)
    y = y * cw

    @pl.when(e == 0)
    def _():
        o_ref[...] = y

    @pl.when(e > 0)
    def _():
        o_ref[...] += y

    @pl.when(e == E - 1)
    def _():
        o_ref[...] = _ln(o_ref[...], lw_ref[...], lb_ref[...])


def _moe(x, ew1_t, eb1, ew2_t, eb2, comb, lw, lb):
    return pl.pallas_call(
        _moe_body,
        grid=(E,),
        in_specs=[
            pl.BlockSpec((S, D), lambda e: (0, 0)),
            pl.BlockSpec((1, D, EFF), lambda e: (e, 0, 0)),
            pl.BlockSpec((1, 1, EFF), lambda e: (e, 0, 0)),
            pl.BlockSpec((1, EFF, D), lambda e: (e, 0, 0)),
            pl.BlockSpec((1, 1, D), lambda e: (e, 0, 0)),
            pl.BlockSpec((S, EPAD), lambda e: (0, 0)),
            pl.BlockSpec((1, D), lambda e: (0, 0)),
            pl.BlockSpec((1, D), lambda e: (0, 0)),
        ],
        out_specs=pl.BlockSpec((S, D), lambda e: (0, 0)),
        out_shape=jax.ShapeDtypeStruct((S, D), jnp.float32),
    )(x, ew1_t, eb1, ew2_t, eb2, comb, lw, lb)


def kernel(input_ids, embed_tokens, embed_positions, in_proj_w, in_proj_b,
           out_proj_w, out_proj_b, ln1_w, ln1_b, lin1_w, lin1_b, lin2_w,
           lin2_b, ln2_w, ln2_b, router_w, expert_w1, expert_b1, expert_w2,
           expert_b2, lnf_w, lnf_b):
    ids = input_ids.reshape(-1).astype(jnp.int32)
    emb = _sc_embed_gather(ids, embed_tokens)

    row = lambda v: v.reshape(1, -1)
    h, qkv = _qkv(emb, embed_positions[:S], in_proj_w, row(in_proj_b))
    hd = lambda t: t.reshape(S, H, DH).transpose(1, 0, 2)
    attn3 = _attention(hd(qkv[:, :D]), hd(qkv[:, D:2 * D]),
                       hd(qkv[:, 2 * D:]))
    attn = attn3.transpose(1, 0, 2).reshape(S, D)
    x = _layer_tail(attn, out_proj_w, row(out_proj_b), h, row(ln1_w),
                    row(ln1_b), lin1_w, row(lin1_b), lin2_w, row(lin2_b),
                    row(ln2_w), row(ln2_b))

    wr_pad_t = jnp.zeros((EPAD, D), jnp.float32).at[:E].set(router_w)
    comb, loss = _router(x, wr_pad_t)

    ew1_t = expert_w1.transpose(0, 2, 1)
    ew2_t = expert_w2.transpose(0, 2, 1)
    hf = _moe(x, ew1_t, expert_b1.reshape(E, 1, EFF), ew2_t,
              expert_b2.reshape(E, 1, D), comb, row(lnf_w), row(lnf_b))
    return hf.reshape(1, S, D), loss[0, 0]


# speedup vs baseline: 1.9607x; 1.0019x over previous
import functools

import jax
import jax.numpy as jnp
from jax import lax
from jax.experimental import pallas as pl
from jax.experimental.pallas import tpu as pltpu
from jax.experimental.pallas import tpu_sc as plsc

S = 2048
D = 768
H = 12
DH = 64
E = 8
FF = 3072
EFF = 1024
EPAD = 128

RB = 512
NRB = S // RB


SC_NC = 2
SC_NS = 16


def _sc_embed_gather(ids, table):
    nw = SC_NC * SC_NS
    bpw = S // nw
    mesh = plsc.VectorSubcoreMesh(core_axis_name="c", subcore_axis_name="s",
                                  num_cores=SC_NC, num_subcores=SC_NS)

    @functools.partial(
        pl.kernel,
        mesh=mesh,
        out_type=jax.ShapeDtypeStruct((S, D), jnp.float32),
        scratch_types=[
            pltpu.VMEM((bpw,), jnp.int32),
            pltpu.VMEM((bpw, D), jnp.float32),
            pltpu.SemaphoreType.DMA,
        ],
    )
    def k(ids_hbm, tab_hbm, out_hbm, idx_v, rows_v, sem):
        wid = lax.axis_index("s") * SC_NC + lax.axis_index("c")
        base = wid * bpw
        pltpu.sync_copy(ids_hbm.at[pl.ds(base, bpw)], idx_v)
        pltpu.async_copy(tab_hbm.at[idx_v], rows_v, sem).wait()
        pltpu.sync_copy(rows_v, out_hbm.at[pl.ds(base, bpw)])

    return k(ids, table)


def _qkv_body(emb_ref, pos_ref, w_ref, b_ref, h_ref, qkv_ref):
    h = emb_ref[...] + pos_ref[...]
    h_ref[...] = h
    qkv_ref[...] = lax.dot_general(
        h, w_ref[...], (((1,), (1,)), ((), ())),
        preferred_element_type=jnp.float32) + b_ref[...]


def _qkv(emb, pos, w_t, b):
    return pl.pallas_call(
        _qkv_body,
        grid=(NRB,),
        in_specs=[
            pl.BlockSpec((RB, D), lambda i: (i, 0)),
            pl.BlockSpec((RB, D), lambda i: (i, 0)),
            pl.BlockSpec((3 * D, D), lambda i: (0, 0)),
            pl.BlockSpec((1, 3 * D), lambda i: (0, 0)),
        ],
        out_specs=[
            pl.BlockSpec((RB, D), lambda i: (i, 0)),
            pl.BlockSpec((RB, 3 * D), lambda i: (i, 0)),
        ],
        out_shape=[
            jax.ShapeDtypeStruct((S, D), jnp.float32),
            jax.ShapeDtypeStruct((S, 3 * D), jnp.float32),
        ],
    )(emb, pos, w_t, b)


ACH = 1024


def _attn_body(q_ref, k_ref, v_ref, o_ref):
    q = q_ref[0]
    m = jnp.full((S, 1), -jnp.inf, jnp.float32)
    den = jnp.zeros((S, 1), jnp.float32)
    acc = jnp.zeros((S, DH), jnp.float32)
    for c in range(S // ACH):
        kc = k_ref[0, c * ACH:(c + 1) * ACH, :]
        vc = v_ref[0, c * ACH:(c + 1) * ACH, :]
        s = lax.dot_general(q, kc, (((1,), (1,)), ((), ())),
                            preferred_element_type=jnp.float32)
        s = s * (1.0 / (DH ** 0.5))
        mn = jnp.maximum(m, jnp.max(s, axis=-1, keepdims=True))
        alpha = jnp.exp(m - mn)
        e = jnp.exp(s - mn)
        den = den * alpha + jnp.sum(e, axis=-1, keepdims=True)
        acc = acc * alpha + jnp.dot(e, vc, preferred_element_type=jnp.float32)
        m = mn
    o_ref[0] = acc / den


def _attention(q3, k3, v3):
    return pl.pallas_call(
        _attn_body,
        grid=(H,),
        in_specs=[
            pl.BlockSpec((1, S, DH), lambda h: (h, 0, 0)),
            pl.BlockSpec((1, S, DH), lambda h: (h, 0, 0)),
            pl.BlockSpec((1, S, DH), lambda h: (h, 0, 0)),
        ],
        out_specs=pl.BlockSpec((1, S, DH), lambda h: (h, 0, 0)),
        out_shape=jax.ShapeDtypeStruct((H, S, DH), jnp.float32),
    )(q3, k3, v3)


def _ln(x, w, b):
    m = jnp.mean(x, axis=-1, keepdims=True)
    xc = x - m
    v = jnp.mean(xc * xc, axis=-1, keepdims=True)
    return xc / jnp.sqrt(v + 1e-5) * w + b


def _tail_body(a_ref, w_ref, b_ref, h_ref, l1w_ref, l1b_ref, w1_ref, b1_ref,
               w2_ref, b2_ref, l2w_ref, l2b_ref, o_ref):
    o = lax.dot_general(
        a_ref[...], w_ref[...], (((1,), (1,)), ((), ())),
        preferred_element_type=jnp.float32) + b_ref[...]
    h1 = _ln(h_ref[...] + o, l1w_ref[...], l1b_ref[...])
    mid = jnp.maximum(
        lax.dot_general(h1, w1_ref[...], (((1,), (1,)), ((), ())),
                        preferred_element_type=jnp.float32)
        + b1_ref[...], 0.0)
    ff = lax.dot_general(mid, w2_ref[...], (((1,), (1,)), ((), ())),
                         preferred_element_type=jnp.float32) + b2_ref[...]
    o_ref[...] = _ln(h1 + ff, l2w_ref[...], l2b_ref[...])


def _layer_tail(attn, w_t, b, h, l1w, l1b, w1_t, b1, w2_t, b2, l2w, l2b):
    rowspec = pl.BlockSpec((1, D), lambda i: (0, 0))
    return pl.pallas_call(
        _tail_body,
        grid=(NRB,),
        in_specs=[
            pl.BlockSpec((RB, D), lambda i: (i, 0)),
            pl.BlockSpec((D, D), lambda i: (0, 0)),
            rowspec,
            pl.BlockSpec((RB, D), lambda i: (i, 0)),
            rowspec,
            rowspec,
            pl.BlockSpec((FF, D), lambda i: (0, 0)),
            pl.BlockSpec((1, FF), lambda i: (0, 0)),
            pl.BlockSpec((D, FF), lambda i: (0, 0)),
            rowspec,
            rowspec,
            rowspec,
        ],
        out_specs=pl.BlockSpec((RB, D), lambda i: (i, 0)),
        out_shape=jax.ShapeDtypeStruct((S, D), jnp.float32),
    )(attn, w_t, b, h, l1w, l1b, w1_t, b1, w2_t, b2, l2w, l2b)


def _router_body(x_ref, w_ref, comb_ref, loss_ref):
    lp = lax.dot_general(x_ref[...], w_ref[...], (((1,), (1,)), ((), ())),
                         preferred_element_type=jnp.float32)
    col = lax.broadcasted_iota(jnp.int32, (S, EPAD), 1)
    valid = col < E
    neg = jnp.float32(-1e30)
    lpm = jnp.where(valid, lp, neg)
    m = jnp.max(lpm, axis=-1, keepdims=True)
    el = jnp.where(valid, jnp.exp(lpm - m), 0.0)
    denom = jnp.sum(el, axis=-1, keepdims=True)
    p = el / denom
    m1 = jnp.max(p, axis=-1, keepdims=True)
    i1 = jnp.min(jnp.where((p == m1) & valid, col, EPAD), axis=-1,
                 keepdims=True)
    p2 = jnp.where(col == i1, -1.0, p)
    m2 = jnp.max(p2, axis=-1, keepdims=True)
    i2 = jnp.min(jnp.where((p2 == m2) & valid, col, EPAD), axis=-1,
                 keepdims=True)
    wsum = m1 + m2
    sel1 = (col == i1).astype(jnp.float32)
    sel2 = (col == i2).astype(jnp.float32)
    comb_ref[...] = (sel1 * (m1 / wsum) + sel2 * (m2 / wsum))
    counts = jnp.sum(sel1 + sel2, axis=0, keepdims=True)
    load = counts / jnp.float32(S * 2)
    dev = jnp.where(col[:1] < E, load - 1.0 / E, 0.0)
    aux = jnp.sum(dev * dev) / jnp.float32(E)
    zrow = jnp.log(denom) + m
    zl = jnp.sum(zrow) / jnp.float32(S)
    loss_ref[...] = jnp.full((8, 128), 0.01 * aux + 0.001 * zl,
                             dtype=jnp.float32)


def _moe_body(x_ref, wr_ref, w1_ref, b1_ref, w2_ref, b2_ref, lw_ref,
              lb_ref, o_ref, loss_ref, comb_ref):
    e = pl.program_id(0)
    x = x_ref[...]

    @pl.when(e == 0)
    def _():
        _router_body(x_ref, wr_ref, comb_ref, loss_ref)
    mid = jnp.dot(x, w1_ref[0], preferred_element_type=jnp.float32) \
        + b1_ref[0]
    mid = 0.5 * mid * (1.0 + lax.erf(mid * (2.0 ** -0.5)))
    y = jnp.dot(mid, w2_ref[0], preferred_element_type=jnp.float32) \
        + b2_ref[0]
    col = lax.broadcasted_iota(jnp.int32, (S, EPAD), 1)
    cw = jnp.sum(jnp.where(col == e, comb_ref[...], 0.0), axis=-1,
                 keepdims=True)
    y = y * cw

    @pl.when(e == 0)
    def _():
        o_ref[...] = y

    @pl.when(e > 0)
    def _():
        o_ref[...] += y

    @pl.when(e == E - 1)
    def _():
        o_ref[...] = _ln(o_ref[...], lw_ref[...], lb_ref[...])


def _moe(x, wr_pad, ew1_t, eb1, ew2_t, eb2, lw, lb):
    return pl.pallas_call(
        _moe_body,
        grid=(E,),
        in_specs=[
            pl.BlockSpec((S, D), lambda e: (0, 0)),
            pl.BlockSpec((EPAD, D), lambda e: (0, 0)),
            pl.BlockSpec((1, D, EFF), lambda e: (e, 0, 0)),
            pl.BlockSpec((1, 1, EFF), lambda e: (e, 0, 0)),
            pl.BlockSpec((1, EFF, D), lambda e: (e, 0, 0)),
            pl.BlockSpec((1, 1, D), lambda e: (e, 0, 0)),
            pl.BlockSpec((1, D), lambda e: (0, 0)),
            pl.BlockSpec((1, D), lambda e: (0, 0)),
        ],
        out_specs=[
            pl.BlockSpec((S, D), lambda e: (0, 0)),
            pl.BlockSpec((8, 128), lambda e: (0, 0)),
        ],
        out_shape=[
            jax.ShapeDtypeStruct((S, D), jnp.float32),
            jax.ShapeDtypeStruct((8, 128), jnp.float32),
        ],
        scratch_shapes=[pltpu.VMEM((S, EPAD), jnp.float32)],
    )(x, wr_pad, ew1_t, eb1, ew2_t, eb2, lw, lb)


def kernel(input_ids, embed_tokens, embed_positions, in_proj_w, in_proj_b,
           out_proj_w, out_proj_b, ln1_w, ln1_b, lin1_w, lin1_b, lin2_w,
           lin2_b, ln2_w, ln2_b, router_w, expert_w1, expert_b1, expert_w2,
           expert_b2, lnf_w, lnf_b):
    ids = input_ids.reshape(-1).astype(jnp.int32)
    emb = _sc_embed_gather(ids, embed_tokens)

    row = lambda v: v.reshape(1, -1)
    h, qkv = _qkv(emb, embed_positions[:S], in_proj_w, row(in_proj_b))
    hd = lambda t: t.reshape(S, H, DH).transpose(1, 0, 2)
    attn3 = _attention(hd(qkv[:, :D]), hd(qkv[:, D:2 * D]),
                       hd(qkv[:, 2 * D:]))
    attn = attn3.transpose(1, 0, 2).reshape(S, D)
    x = _layer_tail(attn, out_proj_w, row(out_proj_b), h, row(ln1_w),
                    row(ln1_b), lin1_w, row(lin1_b), lin2_w, row(lin2_b),
                    row(ln2_w), row(ln2_b))

    wr_pad = jnp.zeros((EPAD, D), jnp.float32).at[:E].set(router_w)
    ew1_t = expert_w1.transpose(0, 2, 1)
    ew2_t = expert_w2.transpose(0, 2, 1)
    hf, loss = _moe(x, wr_pad, ew1_t, expert_b1.reshape(E, 1, EFF), ew2_t,
                    expert_b2.reshape(E, 1, D), row(lnf_w), row(lnf_b))
    return hf.reshape(1, S, D), loss[0, 0]
